# baseline (speedup 1.0000x reference)
import jax
import jax.numpy as jnp
from jax.experimental import pallas as pl
from jax.experimental.pallas import tpu as pltpu

EPS = 1e-5
B = 1024
NPF, NSV = 50, 5
NV = 55
NP = 56
K = 7
M = K * NP
F32 = jnp.float32
NK = B * NV * K
N1 = B * NV


def _row_mask(rows):
    n = jax.lax.broadcasted_iota(jnp.int32, (1, rows, 1), 1) % NP
    return (n < NV).astype(F32)


def _acc(ref, val, i):
    @pl.when(i == 0)
    def _():
        ref[...] = jnp.zeros_like(ref)
    ref[...] += val


def _r(x):
    return x.astype(jnp.bfloat16).astype(F32)


def _dot(a, b):
    return jax.lax.dot_general(a, b, (((1,), (1,)), ((), ())),
                               preferred_element_type=F32)


def _mom(a, b):
    return jax.lax.dot_general(a, b, (((0,), (0,)), ((), ())),
                               preferred_element_type=F32)


def _s8(x2d):
    r, c = x2d.shape
    return jnp.sum(x2d.reshape(r // 8, 8, c), axis=0)


def _colsum(x):
    return jnp.sum(x, axis=0, keepdims=True)


def _diag(S):
    eye = (jax.lax.broadcasted_iota(jnp.int32, S.shape, 0)
           == jax.lax.broadcasted_iota(jnp.int32, S.shape, 1)).astype(F32)
    return jnp.sum(S * eye, axis=0)[None, :]


def _zstats(wa, wb, sff, sfn, snn, muf, mun, isv):
    if isv is not None:
        ss = isv.T * isv
        e11 = (sff - muf.T * muf) * ss
        e12 = (sfn - sff - muf.T * mun + muf.T * muf) * ss
        e22 = (snn - sfn - sfn.T + sff) * ss
        m1 = jnp.zeros_like(muf)
        m2 = (mun - muf) * isv
    else:
        e11 = sff
        e12 = sfn - sff
        e22 = snn - sfn - sfn.T + sff
        m1 = muf
        m2 = mun - muf
    mean = _dot(m1, wa) + _dot(m2, wb)
    d1 = jnp.sum((wa @ e11) * wa, axis=1)[None, :]
    d2 = jnp.sum((wa @ e12) * wb, axis=1)[None, :]
    d4 = jnp.sum((wb @ e22) * wb, axis=1)[None, :]
    var = d1 + 2.0 * d2 + d4 - mean * mean
    return mean, jax.lax.rsqrt(var + EPS)


def _lin_stats(w, S, mu):
    mean = _dot(mu, w)
    e2 = jnp.sum((w @ S) * w, axis=1)[None, :]
    return mean, jax.lax.rsqrt(e2 - mean * mean + EPS)


def _unpack_gm(gm, c):
    A = gm[0:c, :] / NK
    Bm = gm[c:2 * c, :] / NK
    D = gm[2 * c:3 * c, :] / NK
    muf = _colsum(gm[3 * c:3 * c + 8, :]) / NK
    mun = _colsum(gm[3 * c + 8:3 * c + 16, :]) / NK
    return A, Bm, D, muf, mun


def _unpack_gh(gh, c):
    S = gh[0:c, :] / NK
    mu = _colsum(gh[c:c + 8, :]) / NK
    return S, mu


def _knn_idx(d2):
    iota_n = jax.lax.broadcasted_iota(jnp.int32, d2.shape, 1)
    iota_m = jax.lax.broadcasted_iota(jnp.int32, d2.shape, 2)
    bits = jax.lax.bitcast_convert_type(jnp.maximum(d2, 0.0), jnp.int32)
    big = jnp.int32(0x7FFFFFFF)
    bits = jnp.where((iota_n == iota_m) | (iota_m >= NV), big, bits)
    sels = []
    for _ in range(K):
        cmin = jnp.min(bits, axis=2)
        hit = bits == cmin[:, :, None]
        sel = jnp.min(jnp.where(hit, iota_m, big), axis=2)
        sels.append(sel)
        bits = jnp.where(iota_m == sel[:, :, None], big, bits)
    return jnp.concatenate(sels, axis=1)


def _gather_cl(table_cl, idxlane, cdim):
    bb = table_cl.shape[0]
    tcf = jnp.swapaxes(table_cl, 1, 2)
    idxb = jnp.broadcast_to(idxlane[:, None, :], (bb, cdim, M))
    return jnp.swapaxes(jnp.take_along_axis(tcf, idxb, axis=2), 1, 2)


def _rep7(u):
    return jnp.concatenate([u] * K, axis=1)


def _ksum(x3, c):
    return sum(x3[:, k * NP:(k + 1) * NP, :] for k in range(K))


def _gather_moments(table, nbm, gm_ref, i):
    bb, _, c = table.shape
    t2 = table.reshape(bb * NP, c)
    nb2 = nbm.reshape(bb * M, c)
    nbsum2 = _ksum(nbm, c).reshape(bb * NP, c)
    A = 7.0 * _mom(t2, t2)
    Bm = _mom(t2, nbsum2)
    D = _mom(nb2, nb2)
    val = jnp.concatenate([A, Bm, D, 7.0 * _s8(t2), _s8(nb2)], axis=0)
    _acc(gm_ref, val, i)


def _h_moments(hm2, gh_ref, i):
    val = jnp.concatenate([_mom(hm2, hm2), _s8(hm2)], axis=0)
    _acc(gh_ref, val, i)


def _k1(pf_ref, sv_ref, pfs_ref, pfS_ref, svs_ref, svS_ref):
    i = pl.program_id(0)
    for x_ref, s_ref, S_ref, cdim, ndim in ((pf_ref, pfs_ref, pfS_ref, 22, NPF),
                                            (sv_ref, svs_ref, svS_ref, 12, NSV)):
        bb = x_ref.shape[0]
        xt = jnp.swapaxes(x_ref[...], 1, 2)
        x2 = xt.reshape(bb * ndim, cdim)
        _acc(s_ref, jnp.sum(x2, axis=0)[None, :], i)
        _acc(S_ref, _mom(x2, x2), i)


def _feature_conv(x_ref, s_ref, S_ref, w_ref, cdim, ndim):
    n_tot = B * ndim
    S = S_ref[...] / n_tot
    m = s_ref[...] / n_tot
    var1 = _diag(S) - m * m
    inv1 = jax.lax.rsqrt(var1 + EPS)
    C = (S - m.T * m) * inv1.T * inv1
    w = w_ref[...]
    var2 = jnp.sum((w @ C) * w, axis=1)[None, :]
    inv2 = jax.lax.rsqrt(var2 + EPS)
    bb = x_ref.shape[0]
    xt = jnp.swapaxes(x_ref[...], 1, 2)
    f1 = (xt - m[None, :, :]) * inv1[None, :, :]
    y = _dot(_r(f1.reshape(bb * ndim, cdim)), _r(w))
    return jax.nn.relu(y * inv2).reshape(bb, ndim, 32)


def _k2(pf_ref, sv_ref, pfp_ref, svp_ref, wpf_ref, wsv_ref,
        pfs_ref, pfS_ref, svs_ref, svS_ref,
        feat_ref, nb_ref, gm_ref):
    i = pl.program_id(0)
    p1 = _feature_conv(pf_ref, pfs_ref, pfS_ref, wpf_ref, 22, NPF)
    p2 = _feature_conv(sv_ref, svs_ref, svS_ref, wsv_ref, 12, NSV)
    bb = p1.shape[0]
    feat = jnp.concatenate([p1, p2, jnp.zeros((bb, 1, 32), F32)], axis=1)
    feat_ref[...] = feat
    pts = jnp.concatenate([pfp_ref[...], svp_ref[...],
                           jnp.zeros((bb, 2, 1), F32)], axis=2)
    px, py = pts[:, 0, :], pts[:, 1, :]
    xx = px * px + py * py
    pxr = px.astype(jnp.bfloat16).astype(F32)
    pyr = py.astype(jnp.bfloat16).astype(F32)
    dot2 = 2.0 * (pxr[:, :, None] * pxr[:, None, :]
                  + pyr[:, :, None] * pyr[:, None, :])
    nd = (-xx[:, :, None] + dot2) - xx[:, None, :]
    idxlane = _knn_idx(-nd)
    nbm = _gather_cl(feat, idxlane, 32) * _row_mask(M)
    nb_ref[...] = nbm
    _gather_moments(feat, nbm, gm_ref, i)


def _ec_chain(depth, feat, nb, ws, gm, gh_list, tbl_mv):
    bb, _, ci = feat.shape
    A, Bm, D, muf, mun = _unpack_gm(gm, ci)
    w0 = ws[0]
    wa, wb = w0[:, :ci], w0[:, ci:]
    if tbl_mv is not None:
        mu_t, inv_t = tbl_mv
        x1t = (feat - mu_t[None, :, :]) * inv_t[None, :, :]
        isv = inv_t
    else:
        x1t = feat
        isv = None
    mean, inv = _zstats(wa, wb, A, Bm, D, muf, mun, isv)
    x1r = _rep7(x1t)
    if tbl_mv is not None:
        x2 = (nb - mu_t[None, :, :]) * inv_t[None, :, :] - x1r
    else:
        x2 = nb - x1r
    z = (_rep7(_dot(_r(x1t.reshape(bb * NP, ci)), _r(wa)).reshape(bb, NP, -1))
         + _dot(_r(x2.reshape(bb * M, ci)), _r(wb)).reshape(bb, M, -1))
    h = jax.nn.relu((z - mean[None, :, :]) * inv[None, :, :])
    for d in range(depth):
        co = ws[d + 1].shape[0]
        S, mu = _unpack_gh(gh_list[d], h.shape[2])
        mean, inv = _lin_stats(ws[d + 1], S, mu)
        z = _dot(_r(h.reshape(bb * M, h.shape[2])),
                 _r(ws[d + 1])).reshape(bb, M, co)
        h = jax.nn.relu((z - mean[None, :, :]) * inv[None, :, :])
    return h


def _tbl_inv(gm, c):
    A, _, _, muf, _ = _unpack_gm(gm, c)
    return muf, jax.lax.rsqrt(_diag(A) - muf * muf + EPS)


def _make_mid_kernel(depth, normalize_table):
    def kern(feat_ref, nb_ref, gm_ref, *rest):
        i = pl.program_id(0)
        nw = depth + 1
        ws = [rest[j][...] for j in range(nw)]
        ghs = [rest[nw + j][...] for j in range(depth)]
        out_ref = rest[nw + depth]
        gm = gm_ref[...]
        feat, nb = feat_ref[...], nb_ref[...]
        tbl_mv = _tbl_inv(gm, feat.shape[2]) if normalize_table else None
        h = _ec_chain(depth, feat, nb, ws, gm, ghs, tbl_mv)
        bb, _, co = h.shape
        hm2 = (h * _row_mask(M)).reshape(bb * M, co)
        _h_moments(hm2, out_ref, i)
    return kern


def _k6(feat_ref, nb_ref, gm_ref, w0_ref, w1_ref, w2_ref,
        g1_ref, g2_ref,
        out1_ref, nb2_ref, gm2_ref):
    i = pl.program_id(0)
    gm = gm_ref[...]
    feat, nb = feat_ref[...], nb_ref[...]
    bb = feat.shape[0]
    muf, inv_tbl = _tbl_inv(gm, 32)
    ws = [w0_ref[...], w1_ref[...], w2_ref[...]]
    h3 = _ec_chain(2, feat, nb, ws, gm, [g1_ref[...], g2_ref[...]],
                   (muf, inv_tbl))
    mk = _ksum(h3, 32) * (1.0 / K)
    mask56 = _row_mask(NP)
    fts0 = (feat - muf[None, :, :]) * inv_tbl[None, :, :]
    out1 = jax.nn.relu(fts0 + mk) * mask56
    out1_ref[...] = out1
    xx = jnp.sum(out1 * out1, axis=2)
    o1r = out1.astype(jnp.bfloat16).astype(F32)
    g = jax.lax.dot_general(o1r, jnp.swapaxes(o1r, 1, 2),
                            (((2,), (1,)), ((0,), (0,))),
                            preferred_element_type=F32)
    nd = (-xx[:, :, None] + 2.0 * g) - xx[:, None, :]
    idxlane = _knn_idx(-nd)
    nbm2 = _gather_cl(out1, idxlane, 32) * _row_mask(M)
    nb2_ref[...] = nbm2
    _gather_moments(out1, nbm2, gm2_ref, i)


def _k10(out1_ref, nb2_ref, gm2_ref, w0_ref, w1_ref, w2_ref, scw_ref,
         g1_ref, g2_ref,
         out2_ref, gf_ref):
    i = pl.program_id(0)
    gm2 = gm2_ref[...]
    out1, nb2 = out1_ref[...], nb2_ref[...]
    bb = out1.shape[0]
    ws = [w0_ref[...], w1_ref[...], w2_ref[...]]
    h3 = _ec_chain(2, out1, nb2, ws, gm2, [g1_ref[...], g2_ref[...]], None)
    mk = _ksum(h3, 64) * (1.0 / K)
    A2, _, _, mu1, _ = _unpack_gm(gm2, 32)
    scw = scw_ref[...]
    mean_sc, inv_sc = _lin_stats(scw, A2, mu1)
    sc = (_dot(_r(out1.reshape(bb * NP, 32)), _r(scw)).reshape(bb, NP, 64)
          - mean_sc[None, :, :]) * inv_sc[None, :, :]
    mask56 = _row_mask(NP)
    out2 = jax.nn.relu(sc + mk) * mask56
    out2_ref[...] = out2
    o12, o22 = out1.reshape(bb * NP, 32), out2.reshape(bb * NP, 64)
    val = jnp.concatenate([_mom(o12, o22), _mom(o22, o22), _s8(o22)], axis=0)
    _acc(gf_ref, val, i)


def _k11(out1_ref, out2_ref, gm2_ref, gf_ref, fw_ref,
         fc1w_ref, fc1b_ref, fc2w_ref, fc2b_ref, o_ref):
    out1, out2 = out1_ref[...], out2_ref[...]
    bb = out1.shape[0]
    gm2, gf = gm2_ref[...], gf_ref[...]
    A2, _, _, mu1, _ = _unpack_gm(gm2, 32)
    S11 = A2
    S12 = gf[0:32, :] / N1
    S22 = gf[32:96, :] / N1
    mu2 = _colsum(gf[96:104, :]) / N1
    fw = fw_ref[...]
    fa, fb = fw[:, :32], fw[:, 32:]
    mean = _dot(mu1, fa) + _dot(mu2, fb)
    d1 = jnp.sum((fa @ S11) * fa, axis=1)[None, :]
    d2 = jnp.sum((fa @ S12) * fb, axis=1)[None, :]
    d4 = jnp.sum((fb @ S22) * fb, axis=1)[None, :]
    inv = jax.lax.rsqrt(d1 + 2.0 * d2 + d4 - mean * mean + EPS)
    fr = (_dot(_r(out1.reshape(bb * NP, 32)), _r(fa))
          + _dot(_r(out2.reshape(bb * NP, 64)), _r(fb))).reshape(bb, NP, 128)
    fused = jax.nn.relu((fr - mean[None, :, :]) * inv[None, :, :]) * _row_mask(NP)
    pooled = jnp.sum(fused, axis=1) * (1.0 / NV)
    x1 = jax.nn.relu(_dot(_r(pooled), _r(fc1w_ref[...])) + fc1b_ref[...])
    o_ref[...] = _dot(_r(x1), _r(fc2w_ref[...])) + fc2b_ref[...]


def _spec(shape, blocked_dim0=True):
    if blocked_dim0:
        zeros = (0,) * (len(shape) - 1)
        return pl.BlockSpec(shape, lambda i: (i,) + zeros)
    return pl.BlockSpec(shape, lambda i: (0,) * len(shape))


def _full(shape):
    return _spec(shape, blocked_dim0=False)


def kernel(pf_points, pf_features, pf_mask, sv_points, sv_features, sv_mask,
           pf_conv_w, sv_conv_w, ec1_w0, ec1_w1, ec1_w2,
           ec2_w0, ec2_w1, ec2_w2, ec2_sc_w, fusion_w,
           fc1_w, fc1_b, fc2_w, fc2_b):

    def call(kern, bb, in_arrays, in_specs, out_shapes, out_specs):
        return pl.pallas_call(
            kern, grid=(B // bb,), in_specs=in_specs,
            out_shape=[jax.ShapeDtypeStruct(s, d) for s, d in out_shapes],
            out_specs=out_specs)(*in_arrays)

    b1 = 128
    pf_s, pf_S, sv_s, sv_S = call(
        _k1, b1,
        [pf_features, sv_features],
        [_spec((b1, 22, NPF)), _spec((b1, 12, NSV))],
        [((1, 22), F32), ((22, 22), F32), ((1, 12), F32), ((12, 12), F32)],
        [_full((1, 22)), _full((22, 22)), _full((1, 12)), _full((12, 12))])

    b2 = 64
    feat, nb1, gm1 = call(
        _k2, b2,
        [pf_features, sv_features, pf_points, sv_points, pf_conv_w, sv_conv_w,
         pf_s, pf_S, sv_s, sv_S],
        [_spec((b2, 22, NPF)), _spec((b2, 12, NSV)), _spec((b2, 2, NPF)),
         _spec((b2, 2, NSV)), _full((32, 22)), _full((32, 12)),
         _full((1, 22)), _full((22, 22)), _full((1, 12)), _full((12, 12))],
        [((B, NP, 32), F32), ((B, M, 32), F32), ((112, 32), F32)],
        [_spec((b2, NP, 32)), _spec((b2, M, 32)), _full((112, 32))])

    b3 = 64
    ec1_ws = [ec1_w0, ec1_w1, ec1_w2]
    ec1_w_specs = [_full((32, 64)), _full((32, 32)), _full((32, 32))]
    ghs1 = []
    for depth in range(2):
        kern = _make_mid_kernel(depth, True)
        (gh,) = call(
            kern, b3,
            [feat, nb1, gm1] + ec1_ws[:depth + 1] + ghs1,
            [_spec((b3, NP, 32)), _spec((b3, M, 32)), _full((112, 32))]
            + ec1_w_specs[:depth + 1] + [_full((40, 32))] * depth,
            [((40, 32), F32)], [_full((40, 32))])
        ghs1.append(gh)

    b6 = 32
    out1, nb2, gm2 = call(
        _k6, b6,
        [feat, nb1, gm1, ec1_w0, ec1_w1, ec1_w2] + ghs1,
        [_spec((b6, NP, 32)), _spec((b6, M, 32)), _full((112, 32)),
         _full((32, 64)), _full((32, 32)), _full((32, 32))]
        + [_full((40, 32))] * 2,
        [((B, NP, 32), F32), ((B, M, 32), F32), ((112, 32), F32)],
        [_spec((b6, NP, 32)), _spec((b6, M, 32)), _full((112, 32))])

    b7 = 64
    ec2_ws = [ec2_w0, ec2_w1, ec2_w2]
    ec2_w_specs = [_full((64, 64))] * 3
    ghs2 = []
    for depth in range(2):
        kern = _make_mid_kernel(depth, False)
        (gh,) = call(
            kern, b7,
            [out1, nb2, gm2] + ec2_ws[:depth + 1] + ghs2,
            [_spec((b7, NP, 32)), _spec((b7, M, 32)), _full((112, 32))]
            + ec2_w_specs[:depth + 1] + [_full((72, 64))] * depth,
            [((72, 64), F32)], [_full((72, 64))])
        ghs2.append(gh)

    b10 = 32
    out2, gf = call(
        _k10, b10,
        [out1, nb2, gm2, ec2_w0, ec2_w1, ec2_w2, ec2_sc_w] + ghs2,
        [_spec((b10, NP, 32)), _spec((b10, M, 32)), _full((112, 32)),
         _full((64, 64)), _full((64, 64)), _full((64, 64)), _full((64, 32))]
        + [_full((72, 64))] * 2,
        [((B, NP, 64), F32), ((104, 64), F32)],
        [_spec((b10, NP, 64)), _full((104, 64))])

    b11 = 128
    (out,) = call(
        _k11, b11,
        [out1, out2, gm2, gf, fusion_w,
         fc1_w, fc1_b.reshape(1, 128), fc2_w, fc2_b.reshape(1, 4)],
        [_spec((b11, NP, 32)), _spec((b11, NP, 64)), _full((112, 32)),
         _full((104, 64)), _full((128, 96)), _full((128, 128)),
         _full((1, 128)), _full((4, 128)), _full((1, 4))],
        [((B, 4), F32)],
        [_spec((b11, 4))])
    return out
